# baseline (speedup 1.0000x reference)
import jax
import jax.numpy as jnp
from jax.experimental import pallas as pl
from jax.experimental.pallas import tpu as pltpu
from jax.experimental.pallas import tpu_sc as plsc


def kernel(input, weight):
    B, S = input.shape
    N, D1, D2 = weight.shape
    D = D1 * D2
    num_indices = B * S

    idx = input.reshape(1, num_indices).astype(jnp.int32)

    WINDOW = 256
    assert num_indices % WINDOW == 0

    vmesh = plsc.VectorSubcoreMesh(
        core_axis_name="core", subcore_axis_name="subcore"
    )

    TBW = 40

    @pl.kernel(
        out_type=jax.ShapeDtypeStruct((N, D), weight.dtype),
        mesh=vmesh,
    )
    def sc_table(w_hbm, t_hbm):
        def body(w_vmem, t_vmem):
            @pl.loop(0, TBW)
            def _(r):
                for s in range(D1):
                    t_vmem[r, pl.ds(s * D2, D2)] = w_vmem[r, s, :]

        pltpu.emit_pipeline(
            body,
            grid=(N // TBW,),
            in_specs=[
                pl.BlockSpec((TBW, D1, D2), index_map=lambda i: (i, 0, 0))
            ],
            out_specs=[
                pl.BlockSpec((TBW, D), index_map=lambda i: (i, 0))
            ],
            core_axis_name=("core", "subcore"),
            dimension_semantics=(pltpu.PARALLEL,),
        )(w_hbm, t_hbm)

    @pl.kernel(
        out_type=jax.ShapeDtypeStruct((num_indices, D), weight.dtype),
        mesh=vmesh,
    )
    def sc_gather(x_hbm, i_hbm, o_hbm):
        def body(i_vmem, o_vmem):
            pltpu.sync_copy(x_hbm.at[i_vmem.at[0]], o_vmem)

        pltpu.emit_pipeline(
            body,
            grid=(num_indices // WINDOW,),
            in_specs=[
                pl.BlockSpec((1, WINDOW), index_map=lambda i: (0, i))
            ],
            out_specs=[
                pl.BlockSpec((WINDOW, D), index_map=lambda i: (i, 0))
            ],
            core_axis_name=("core", "subcore"),
            dimension_semantics=(pltpu.PARALLEL,),
        )(i_hbm, o_hbm)

    BB = 32

    def relayout_body(g_ref, o_ref):
        o_ref[...] = g_ref[...].reshape(BB, S, D1, D2)

    tc_relayout = pl.pallas_call(
        relayout_body,
        grid=(B // BB,),
        in_specs=[
            pl.BlockSpec((BB * S, D), lambda i: (i, 0)),
        ],
        out_specs=pl.BlockSpec((BB, S, D1, D2), lambda i: (i, 0, 0, 0)),
        out_shape=jax.ShapeDtypeStruct((B, S, D1, D2), weight.dtype),
        compiler_params=pltpu.CompilerParams(
            dimension_semantics=("parallel",),
        ),
    )

    table = sc_table(weight)
    gathered = sc_gather(table, idx)
    return tc_relayout(gathered)


# speedup vs baseline: 1.7769x; 1.7769x over previous
import jax
import jax.numpy as jnp
from jax.experimental import pallas as pl
from jax.experimental.pallas import tpu as pltpu
from jax.experimental.pallas import tpu_sc as plsc


def kernel(input, weight):
    B, S = input.shape
    N, D1, D2 = weight.shape
    D = D1 * D2
    num_indices = B * S

    table = weight.reshape(N, D)
    idx = input.reshape(1, num_indices).astype(jnp.int32)

    WINDOW = 256
    assert num_indices % WINDOW == 0

    mesh = plsc.VectorSubcoreMesh(
        core_axis_name="core", subcore_axis_name="subcore"
    )

    @pl.kernel(
        out_type=jax.ShapeDtypeStruct((num_indices, D), weight.dtype),
        mesh=mesh,
    )
    def sc_gather(x_hbm, i_hbm, o_hbm):
        def body(i_vmem, o_vmem):
            pltpu.sync_copy(x_hbm.at[i_vmem.at[0]], o_vmem)

        pltpu.emit_pipeline(
            body,
            grid=(num_indices // WINDOW,),
            in_specs=[
                pl.BlockSpec((1, WINDOW), index_map=lambda i: (0, i))
            ],
            out_specs=[
                pl.BlockSpec((WINDOW, D), index_map=lambda i: (i, 0))
            ],
            core_axis_name=("core", "subcore"),
            dimension_semantics=(pltpu.PARALLEL,),
        )(i_hbm, o_hbm)

    @pl.kernel(
        out_type=jax.ShapeDtypeStruct((num_indices, D1, D2), weight.dtype),
        mesh=mesh,
    )
    def sc_format(g_hbm, o_hbm):
        FW = 32
        UNROLL = 4

        def body(g_vmem, o_vmem):
            o_flat = o_vmem.reshape(FW * D1, D2)

            @pl.loop(0, FW, step=UNROLL)
            def _(r):
                for u in range(UNROLL):
                    for s in range(D1):
                        o_flat[(r + u) * D1 + s, :] = g_vmem[
                            r + u, pl.ds(s * D2, D2)
                        ]

        pltpu.emit_pipeline(
            body,
            grid=(num_indices // FW,),
            in_specs=[
                pl.BlockSpec((FW, D), index_map=lambda i: (i, 0))
            ],
            out_specs=[
                pl.BlockSpec(
                    (FW, D1, D2), index_map=lambda i: (i, 0, 0)
                )
            ],
            core_axis_name=("core", "subcore"),
            dimension_semantics=(pltpu.PARALLEL,),
        )(g_hbm, o_hbm)

    gathered = sc_gather(table, idx)
    out = sc_format(gathered)
    return out.reshape(B, S, D1, D2)
